# baseline (speedup 1.0000x reference)
import functools
import math

import jax
import jax.numpy as jnp
from jax.experimental import pallas as pl
from jax.experimental.pallas import tpu as pltpu

H = 16
F32 = jnp.float32
BF16 = jnp.bfloat16


def _proj_body(k_ref, v_ref, wk_ref, bk_ref, wv_ref, bv_ref, kp_ref, vp_ref):
    kp = jax.lax.dot_general(
        k_ref[...], wk_ref[...], (((1,), (1,)), ((), ())),
        preferred_element_type=F32) + bk_ref[...]
    kp_ref[...] = kp.astype(BF16)
    vp_ref[...] = jax.lax.dot_general(
        v_ref[...], wv_ref[...], (((1,), (1,)), ((), ())),
        preferred_element_type=F32) + bv_ref[...]


def _pass1_body(scale, q_ref, b_ref, wq_ref, bq_ref, kp_ref, ex_ref, den_ref):
    i = pl.program_id(0)
    qp = jax.lax.dot_general(
        q_ref[...], wq_ref[...], (((1,), (1,)), ((), ())),
        preferred_element_type=F32) + bq_ref[...]
    b = b_ref[0, 0, :]
    sp = kp_ref.shape[0]
    seg = jax.lax.broadcasted_iota(jnp.int32, (b.shape[0], sp), 1)
    onehot = (b[:, None] == seg).astype(BF16)
    kx = jnp.dot(onehot, kp_ref[...], preferred_element_type=F32)
    ex = jnp.exp(qp * kx * scale).astype(BF16)
    ex_ref[...] = ex
    partial = jax.lax.dot_general(
        onehot, ex, (((0,), (0,)), ((), ())),
        preferred_element_type=F32)

    @pl.when(i == 0)
    def _init():
        den_ref[...] = partial

    @pl.when(i > 0)
    def _acc():
        den_ref[...] += partial


def _pass2_body(ex_ref, b_ref, vp_ref, den_ref, wo_ref, bo_ref, out_ref):
    den = den_ref[...]
    w = jnp.where(den > 0.0, vp_ref[...] / den, 0.0).astype(BF16)
    b = b_ref[0, 0, :]
    sp = vp_ref.shape[0]
    seg = jax.lax.broadcasted_iota(jnp.int32, (b.shape[0], sp), 1)
    onehot = (b[:, None] == seg).astype(BF16)
    wx = jnp.dot(onehot, w, preferred_element_type=F32)
    y = (ex_ref[...].astype(F32) * wx).astype(BF16)
    out_ref[...] = jax.lax.dot_general(
        y, wo_ref[...], (((1,), (1,)), ((), ())),
        preferred_element_type=F32) + bo_ref[...]


def kernel(q, k, v, batch, Wq, bq, Wk, bk, Wv, bv, Wo, bo):
    n, dm = q.shape
    s = k.shape[0]
    d = dm // H
    scale = 1.0 / math.sqrt(float(d))

    R = 512
    nb = -(-n // R)
    npad = nb * R
    sp = -(-(s + 1) // 128) * 128

    qz = jnp.pad(q, ((0, npad - n), (0, 0))).astype(BF16)
    bz = jnp.pad(batch.astype(jnp.int32), (0, npad - n),
                 constant_values=sp - 1)
    b3 = bz.reshape(nb, 1, R)
    kz = jnp.pad(k, ((0, sp - s), (0, 0))).astype(BF16)
    vz = jnp.pad(v, ((0, sp - s), (0, 0))).astype(BF16)
    bq2, bk2, bv2, bo2 = (x.reshape(1, dm) for x in (bq, bk, bv, bo))

    full = lambda *shape: pl.BlockSpec(shape, lambda i: (0,) * len(shape))

    kp, vp = pl.pallas_call(
        _proj_body,
        grid=(1,),
        in_specs=[full(sp, dm), full(sp, dm), full(dm, dm), full(1, dm),
                  full(dm, dm), full(1, dm)],
        out_specs=[full(sp, dm), full(sp, dm)],
        out_shape=[jax.ShapeDtypeStruct((sp, dm), BF16),
                   jax.ShapeDtypeStruct((sp, dm), F32)],
    )(kz, vz, Wk.astype(BF16), bk2, Wv.astype(BF16), bv2)

    ex, den = pl.pallas_call(
        functools.partial(_pass1_body, scale),
        grid=(nb,),
        in_specs=[
            pl.BlockSpec((R, dm), lambda i: (i, 0)),
            pl.BlockSpec((1, 1, R), lambda i: (i, 0, 0)),
            full(dm, dm), full(1, dm), full(sp, dm),
        ],
        out_specs=[pl.BlockSpec((R, dm), lambda i: (i, 0)), full(sp, dm)],
        out_shape=[jax.ShapeDtypeStruct((npad, dm), BF16),
                   jax.ShapeDtypeStruct((sp, dm), F32)],
        compiler_params=pltpu.CompilerParams(
            dimension_semantics=("arbitrary",)),
    )(qz, b3, Wq.astype(BF16), bq2, kp)

    out = pl.pallas_call(
        _pass2_body,
        grid=(nb,),
        in_specs=[
            pl.BlockSpec((R, dm), lambda i: (i, 0)),
            pl.BlockSpec((1, 1, R), lambda i: (i, 0, 0)),
            full(sp, dm), full(sp, dm), full(dm, dm), full(1, dm),
        ],
        out_specs=pl.BlockSpec((R, dm), lambda i: (i, 0)),
        out_shape=jax.ShapeDtypeStruct((npad, dm), F32),
        compiler_params=pltpu.CompilerParams(
            dimension_semantics=("arbitrary",)),
    )(ex, b3, vp, den, Wo.astype(BF16), bo2)

    return out[:n]


# speedup vs baseline: 1.3687x; 1.3687x over previous
import functools
import math

import jax
import jax.numpy as jnp
from jax.experimental import pallas as pl
from jax.experimental.pallas import tpu as pltpu

H = 16
F32 = jnp.float32


def _proj_body(k_ref, v_ref, wk_ref, bk_ref, wv_ref, bv_ref, kp_ref, vp_ref):
    kp_ref[...] = jax.lax.dot_general(
        k_ref[...], wk_ref[...], (((1,), (1,)), ((), ())),
        preferred_element_type=F32) + bk_ref[...]
    vp_ref[...] = jax.lax.dot_general(
        v_ref[...], wv_ref[...], (((1,), (1,)), ((), ())),
        preferred_element_type=F32) + bv_ref[...]


def _pass1_body(scale, q_ref, b_ref, wq_ref, bq_ref, kp_ref, ex_ref, den_ref):
    i = pl.program_id(0)
    qp = jax.lax.dot_general(
        q_ref[...], wq_ref[...], (((1,), (1,)), ((), ())),
        preferred_element_type=F32) + bq_ref[...]
    b = b_ref[0, 0, :]
    sp = kp_ref.shape[0]
    seg = jax.lax.broadcasted_iota(jnp.int32, (b.shape[0], sp), 1)
    onehot = (b[:, None] == seg).astype(F32)
    kx = jnp.dot(onehot, kp_ref[...], preferred_element_type=F32)
    ex = jnp.exp(qp * kx * scale)
    ex_ref[...] = ex
    partial = jax.lax.dot_general(
        onehot, ex, (((0,), (0,)), ((), ())),
        preferred_element_type=F32)

    @pl.when(i == 0)
    def _init():
        den_ref[...] = partial

    @pl.when(i > 0)
    def _acc():
        den_ref[...] += partial


def _pass2_body(ex_ref, b_ref, vp_ref, den_ref, wo_ref, bo_ref, out_ref):
    den = den_ref[...]
    w = jnp.where(den > 0.0, vp_ref[...] / den, 0.0)
    b = b_ref[0, 0, :]
    sp = vp_ref.shape[0]
    seg = jax.lax.broadcasted_iota(jnp.int32, (b.shape[0], sp), 1)
    onehot = (b[:, None] == seg).astype(F32)
    wx = jnp.dot(onehot, w, preferred_element_type=F32)
    y = ex_ref[...] * wx
    out_ref[...] = jax.lax.dot_general(
        y, wo_ref[...], (((1,), (1,)), ((), ())),
        preferred_element_type=F32) + bo_ref[...]


def kernel(q, k, v, batch, Wq, bq, Wk, bk, Wv, bv, Wo, bo):
    n, dm = q.shape
    s = k.shape[0]
    d = dm // H
    scale = 1.0 / math.sqrt(float(d))

    R = 512
    nb = -(-n // R)
    npad = nb * R
    sp = -(-(s + 1) // 128) * 128

    bz = jnp.pad(batch.astype(jnp.int32), (0, npad - n),
                 constant_values=sp - 1)
    b3 = bz.reshape(nb, 1, R)
    kz = jnp.pad(k, ((0, sp - s), (0, 0)))
    vz = jnp.pad(v, ((0, sp - s), (0, 0)))
    bq2, bk2, bv2, bo2 = (x.reshape(1, dm) for x in (bq, bk, bv, bo))

    full = lambda *shape: pl.BlockSpec(shape, lambda i: (0,) * len(shape))

    kp, vp = pl.pallas_call(
        _proj_body,
        grid=(1,),
        in_specs=[full(sp, dm), full(sp, dm), full(dm, dm), full(1, dm),
                  full(dm, dm), full(1, dm)],
        out_specs=[full(sp, dm), full(sp, dm)],
        out_shape=[jax.ShapeDtypeStruct((sp, dm), F32)] * 2,
    )(kz, vz, Wk, bk2, Wv, bv2)

    ex, den = pl.pallas_call(
        functools.partial(_pass1_body, scale),
        grid=(nb,),
        in_specs=[
            pl.BlockSpec((R, dm), lambda i: (i, 0)),
            pl.BlockSpec((1, 1, R), lambda i: (i, 0, 0)),
            full(dm, dm), full(1, dm), full(sp, dm),
        ],
        out_specs=[pl.BlockSpec((R, dm), lambda i: (i, 0)), full(sp, dm)],
        out_shape=[jax.ShapeDtypeStruct((n, dm), F32),
                   jax.ShapeDtypeStruct((sp, dm), F32)],
        compiler_params=pltpu.CompilerParams(
            dimension_semantics=("arbitrary",)),
    )(q, b3, Wq, bq2, kp)

    out = pl.pallas_call(
        _pass2_body,
        grid=(nb,),
        in_specs=[
            pl.BlockSpec((R, dm), lambda i: (i, 0)),
            pl.BlockSpec((1, 1, R), lambda i: (i, 0, 0)),
            full(sp, dm), full(sp, dm), full(dm, dm), full(1, dm),
        ],
        out_specs=pl.BlockSpec((R, dm), lambda i: (i, 0)),
        out_shape=jax.ShapeDtypeStruct((n, dm), F32),
        compiler_params=pltpu.CompilerParams(
            dimension_semantics=("arbitrary",)),
    )(ex, b3, vp, den, Wo, bo2)

    return out


# speedup vs baseline: 1.4204x; 1.0378x over previous
import functools
import math

import jax
import jax.numpy as jnp
from jax.experimental import pallas as pl
from jax.experimental.pallas import tpu as pltpu

H = 16
F32 = jnp.float32
BF16 = jnp.bfloat16


def _proj_body(k_ref, v_ref, wk_ref, bk_ref, wv_ref, bv_ref, kp_ref, vp_ref):
    kp_ref[...] = (jax.lax.dot_general(
        k_ref[...], wk_ref[...], (((1,), (1,)), ((), ())),
        preferred_element_type=F32) + bk_ref[...]).astype(BF16)
    vp_ref[...] = jax.lax.dot_general(
        v_ref[...], wv_ref[...], (((1,), (1,)), ((), ())),
        preferred_element_type=F32) + bv_ref[...]


def _pass1_body(scale, q_ref, b_ref, wq_ref, bq_ref, kp_ref, ex_ref, den_ref):
    i = pl.program_id(0)
    qp = jax.lax.dot_general(
        q_ref[...], wq_ref[...], (((1,), (1,)), ((), ())),
        preferred_element_type=F32) + bq_ref[...]
    b = b_ref[0, 0, :]
    sp = kp_ref.shape[0]
    seg = jax.lax.broadcasted_iota(jnp.int32, (b.shape[0], sp), 1)
    onehot = (b[:, None] == seg).astype(BF16)
    kx = jnp.dot(onehot, kp_ref[...], preferred_element_type=F32)
    ex = jnp.exp(qp * kx * scale).astype(BF16)
    ex_ref[...] = ex
    partial = jax.lax.dot_general(
        onehot, ex, (((0,), (0,)), ((), ())),
        preferred_element_type=F32)

    @pl.when(i == 0)
    def _init():
        den_ref[...] = partial

    @pl.when(i > 0)
    def _acc():
        den_ref[...] += partial


def _pass2_body(ex_ref, b_ref, vp_ref, den_ref, wo_ref, bo_ref, out_ref):
    den = den_ref[...]
    w = jnp.where(den > 0.0, vp_ref[...] / den, 0.0).astype(BF16)
    b = b_ref[0, 0, :]
    sp = vp_ref.shape[0]
    seg = jax.lax.broadcasted_iota(jnp.int32, (b.shape[0], sp), 1)
    onehot = (b[:, None] == seg).astype(BF16)
    wx = jnp.dot(onehot, w, preferred_element_type=F32)
    y = ex_ref[...].astype(F32) * wx
    out_ref[...] = jax.lax.dot_general(
        y, wo_ref[...], (((1,), (1,)), ((), ())),
        preferred_element_type=F32) + bo_ref[...]


def kernel(q, k, v, batch, Wq, bq, Wk, bk, Wv, bv, Wo, bo):
    n, dm = q.shape
    s = k.shape[0]
    d = dm // H
    scale = 1.0 / math.sqrt(float(d))

    R = 512
    nb = -(-n // R)
    npad = nb * R
    sp = -(-(s + 1) // 128) * 128

    bz = jnp.pad(batch.astype(jnp.int32), (0, npad - n),
                 constant_values=sp - 1)
    b3 = bz.reshape(nb, 1, R)
    kz = jnp.pad(k, ((0, sp - s), (0, 0)))
    vz = jnp.pad(v, ((0, sp - s), (0, 0)))
    bq2, bk2, bv2, bo2 = (x.reshape(1, dm) for x in (bq, bk, bv, bo))

    full = lambda *shape: pl.BlockSpec(shape, lambda i: (0,) * len(shape))

    kp, vp = pl.pallas_call(
        _proj_body,
        grid=(1,),
        in_specs=[full(sp, dm), full(sp, dm), full(dm, dm), full(1, dm),
                  full(dm, dm), full(1, dm)],
        out_specs=[full(sp, dm), full(sp, dm)],
        out_shape=[jax.ShapeDtypeStruct((sp, dm), BF16),
                   jax.ShapeDtypeStruct((sp, dm), F32)],
    )(kz, vz, Wk, bk2, Wv, bv2)

    ex, den = pl.pallas_call(
        functools.partial(_pass1_body, scale),
        grid=(nb,),
        in_specs=[
            pl.BlockSpec((R, dm), lambda i: (i, 0)),
            pl.BlockSpec((1, 1, R), lambda i: (i, 0, 0)),
            full(dm, dm), full(1, dm), full(sp, dm),
        ],
        out_specs=[pl.BlockSpec((R, dm), lambda i: (i, 0)), full(sp, dm)],
        out_shape=[jax.ShapeDtypeStruct((n, dm), BF16),
                   jax.ShapeDtypeStruct((sp, dm), F32)],
        compiler_params=pltpu.CompilerParams(
            dimension_semantics=("arbitrary",)),
    )(q, b3, Wq, bq2, kp)

    out = pl.pallas_call(
        _pass2_body,
        grid=(nb,),
        in_specs=[
            pl.BlockSpec((R, dm), lambda i: (i, 0)),
            pl.BlockSpec((1, 1, R), lambda i: (i, 0, 0)),
            full(sp, dm), full(sp, dm), full(dm, dm), full(1, dm),
        ],
        out_specs=pl.BlockSpec((R, dm), lambda i: (i, 0)),
        out_shape=jax.ShapeDtypeStruct((n, dm), F32),
        compiler_params=pltpu.CompilerParams(
            dimension_semantics=("arbitrary",)),
    )(ex, b3, vp, den, Wo, bo2)

    return out
